# baseline (speedup 1.0000x reference)
import functools
import jax
import jax.numpy as jnp
from jax import lax
from jax.experimental import pallas as pl
from jax.experimental.pallas import tpu as pltpu
from jax.experimental.pallas import tpu_sc as plsc

B, N, D = 4, 8192, 768
NC, NS, L = 2, 16, 16
NW = NC * NS
PPW = N // NW
R = 64
NCH = PPW // R
NV = D // L


def _sc_add(inputs, pos_table):
    mesh = plsc.VectorSubcoreMesh(core_axis_name="c", subcore_axis_name="s")

    @functools.partial(
        pl.kernel,
        out_type=jax.ShapeDtypeStruct((B, N, D), jnp.float32),
        mesh=mesh,
        scratch_types=[
            pltpu.VMEM((R, D), jnp.float32),
            pltpu.VMEM((R, D), jnp.float32),
        ],
    )
    def k(inp_hbm, tab_hbm, out_hbm, tbuf, ibuf):
        wid = lax.axis_index("s") * NC + lax.axis_index("c")
        p_base = wid * PPW

        def chunk(c, carry):
            p0 = p_base + c * R
            pltpu.sync_copy(tab_hbm.at[pl.ds(p0, R)], tbuf)

            def one_batch(b, carry2):
                pltpu.sync_copy(inp_hbm.at[b, pl.ds(p0, R)], ibuf)

                def add_row(r, carry3):
                    for j in range(NV):
                        tv = tbuf[r, pl.ds(j * L, L)]
                        plsc.addupdate(ibuf.at[r, pl.ds(j * L, L)], tv)
                    return carry3

                lax.fori_loop(0, R, add_row, 0)
                pltpu.sync_copy(ibuf, out_hbm.at[b, pl.ds(p0, R)])
                return carry2

            lax.fori_loop(0, B, one_batch, 0)
            return carry

        lax.fori_loop(0, NCH, chunk, 0)

    return k(inputs, pos_table)


def kernel(inputs, pos_table):
    return _sc_add(inputs, pos_table)
